# baseline (speedup 1.0000x reference)
import functools

import jax
import jax.numpy as jnp
from jax import lax
from jax.experimental import pallas as pl
from jax.experimental.pallas import tpu as pltpu
from jax.experimental.pallas import tpu_sc as plsc

GAMMA = 12.0
HIDDEN = 128
ENT_DIM = 256
BATCH = 16384
EMB_RANGE = (12.0 + 2.0) / HIDDEN
PHASE_K = 3.141592653589793 / EMB_RANGE

NUM_WORKERS = 32
BPW = BATCH // NUM_WORKERS
CHUNK = 32
NCHUNK = BPW // CHUNK
DEPTH = 4
NROUND = NCHUNK // DEPTH
NGRP = CHUNK // 16
NDC = HIDDEN // 16
RED_STRIDE = 17

NREL = 1000
RSQRT_MAGIC = 0x5F3759DF


def _sincos_tc_kernel(rel_ref, out_ref):
    ph = rel_ref[...] * PHASE_K
    out_ref[:, :HIDDEN] = jnp.cos(ph)
    out_ref[:, HIDDEN:] = jnp.sin(ph)


_sincos_table = pl.pallas_call(
    _sincos_tc_kernel,
    out_shape=jax.ShapeDtypeStruct((NREL, 2 * HIDDEN), jnp.float32),
)


def _sqrt16(v):
    bits = lax.bitcast_convert_type(v, jnp.int32)
    seed = RSQRT_MAGIC - lax.shift_right_arithmetic(bits, 1)
    y = lax.bitcast_convert_type(seed, jnp.float32)
    half_v = 0.5 * v
    for _ in range(2):
        y = y * (1.5 - half_v * y * y)
    return v * y


def _make_sc_kernel():
    mesh = plsc.VectorSubcoreMesh(core_axis_name="c", subcore_axis_name="s")

    @functools.partial(
        pl.kernel,
        mesh=mesh,
        compiler_params=pltpu.CompilerParams(needs_layout_passes=False),
        out_type=jax.ShapeDtypeStruct((BATCH,), jnp.float32),
        scratch_types=[
            pltpu.VMEM((BPW,), jnp.int32),
            pltpu.VMEM((BPW,), jnp.int32),
            pltpu.VMEM((BPW,), jnp.int32),
        ] + [
            pltpu.VMEM((CHUNK, ENT_DIM), jnp.float32)
            for _ in range(3 * DEPTH)
        ] + [
            pltpu.VMEM((CHUNK,), jnp.float32),
            pltpu.VMEM((16 * RED_STRIDE,), jnp.float32),
            pltpu.SemaphoreType.DMA,
            pltpu.SemaphoreType.DMA,
            pltpu.SemaphoreType.DMA,
        ] + [
            pltpu.SemaphoreType.DMA
            for _ in range(3 * DEPTH)
        ],
    )
    def rotate_score(sample_hbm, et_hbm, ent_hbm, rel_hbm, out_hbm,
                     hidx_all, tidx_all, eidx_all,
                     head0, tail0, rel0, head1, tail1, rel1,
                     head2, tail2, rel2, head3, tail3, rel3,
                     out_v, red_v, sem_ih, sem_it, sem_ir,
                     sem_h0, sem_t0, sem_r0, sem_h1, sem_t1, sem_r1,
                     sem_h2, sem_t2, sem_r2, sem_h3, sem_t3, sem_r3):
        wid = lax.axis_index("s") * 2 + lax.axis_index("c")
        base = wid * BPW
        lane_iota = lax.broadcasted_iota(jnp.int32, (16,), 0)
        col_base = lane_iota * RED_STRIDE

        ci = pltpu.async_copy(
            sample_hbm.at[0, pl.ds(base, BPW)], hidx_all, sem_ih)
        ct = pltpu.async_copy(
            sample_hbm.at[1, pl.ds(base, BPW)], tidx_all, sem_it)
        ce = pltpu.async_copy(et_hbm.at[pl.ds(base, BPW)], eidx_all, sem_ir)
        ci.wait()
        ct.wait()
        ce.wait()

        bufs = (
            (head0, tail0, rel0, sem_h0, sem_t0, sem_r0),
            (head1, tail1, rel1, sem_h1, sem_t1, sem_r1),
            (head2, tail2, rel2, sem_h2, sem_t2, sem_r2),
            (head3, tail3, rel3, sem_h3, sem_t3, sem_r3),
        )

        def issue(c, b):
            head_v, tail_v, rel_v, sh, st, sr = bufs[b]
            co = c * CHUNK
            pltpu.async_copy(
                ent_hbm.at[hidx_all.at[pl.ds(co, CHUNK)]], head_v, sh)
            pltpu.async_copy(
                ent_hbm.at[tidx_all.at[pl.ds(co, CHUNK)]], tail_v, st)
            pltpu.async_copy(
                rel_hbm.at[eidx_all.at[pl.ds(co, CHUNK)]], rel_v, sr)

        def drain(b):
            head_v, tail_v, rel_v, sh, st, sr = bufs[b]
            pltpu.make_async_copy(
                ent_hbm.at[pl.ds(0, CHUNK)], head_v, sh).wait()
            pltpu.make_async_copy(
                ent_hbm.at[pl.ds(0, CHUNK)], tail_v, st).wait()
            pltpu.make_async_copy(
                rel_hbm.at[pl.ds(0, CHUNK)], rel_v, sr).wait()

        def compute(c, b):
            head_v, tail_v, rel_v, _, _, _ = bufs[b]
            cb = base + c * CHUNK

            def group_body(g, carry1):
                def elem_body(e, carry2):
                    i = g * 16 + e
                    acc = jnp.zeros((16,), jnp.float32)
                    for dc in range(NDC):
                        sl = pl.ds(dc * 16, 16)
                        sl_im = pl.ds(HIDDEN + dc * 16, 16)
                        cos_r = rel_v[i, sl]
                        sin_r = rel_v[i, sl_im]
                        re_h = head_v[i, sl]
                        im_h = head_v[i, sl_im]
                        re_s = re_h * cos_r - im_h * sin_r - tail_v[i, sl]
                        im_s = re_h * sin_r + im_h * cos_r - tail_v[i, sl_im]
                        acc = acc + _sqrt16(re_s * re_s + im_s * im_s)
                    plsc.store_scatter(red_v, [lane_iota + e * RED_STRIDE],
                                       acc)
                    return carry2

                lax.fori_loop(0, 16, elem_body, 0)
                tot = jnp.zeros((16,), jnp.float32)
                for cc in range(16):
                    tot = tot + plsc.load_gather(red_v, [col_base + cc])
                sc = GAMMA - tot
                u = jnp.exp(-jnp.abs(sc))
                t = u / (u + 2.0)
                t2 = t * t
                log1p = 2.0 * t * (1.0 + t2 * (1.0 / 3.0 + t2 * (
                    1.0 / 5.0 + t2 * (1.0 / 7.0))))
                out_v[pl.ds(g * 16, 16)] = jnp.minimum(sc, 0.0) - log1p
                return carry1

            lax.fori_loop(0, NGRP, group_body, 0)
            pltpu.sync_copy(out_v, out_hbm.at[pl.ds(cb, CHUNK)])

        issue(0, 0)
        issue(1, 1)
        issue(2, 2)

        def round_body(k, carry):
            c0 = DEPTH * k
            issue(c0 + 3, 3)
            drain(0)
            compute(c0, 0)
            issue(c0 + 4, 0)
            drain(1)
            compute(c0 + 1, 1)
            issue(c0 + 5, 1)
            drain(2)
            compute(c0 + 2, 2)
            issue(c0 + 6, 2)
            drain(3)
            compute(c0 + 3, 3)
            return carry

        lax.fori_loop(0, NROUND - 1, round_body, 0)
        c0 = NCHUNK - DEPTH
        issue(c0 + 3, 3)
        drain(0)
        compute(c0, 0)
        drain(1)
        compute(c0 + 1, 1)
        drain(2)
        compute(c0 + 2, 2)
        drain(3)
        compute(c0 + 3, 3)

    return rotate_score


_SC_KERNEL = _make_sc_kernel()


@jax.jit
def kernel(sample, et, entity_embedding, relation_embedding):
    sincos = _sincos_table(relation_embedding)
    return _SC_KERNEL(sample, et, entity_embedding, sincos)


# speedup vs baseline: 1.0167x; 1.0167x over previous
import functools

import jax
import jax.numpy as jnp
from jax import lax
from jax.experimental import pallas as pl
from jax.experimental.pallas import tpu as pltpu
from jax.experimental.pallas import tpu_sc as plsc

GAMMA = 12.0
HIDDEN = 128
ENT_DIM = 256
BATCH = 16384
EMB_RANGE = (12.0 + 2.0) / HIDDEN
PHASE_K = 3.141592653589793 / EMB_RANGE

NUM_WORKERS = 32
BPW = BATCH // NUM_WORKERS
CHUNK = 64
NCHUNK = BPW // CHUNK
NPAIR = NCHUNK // 2
NGRP = CHUNK // 16
NDC = HIDDEN // 16
RED_STRIDE = 17

NREL = 1000
RSQRT_MAGIC = 0x5F3759DF


def _sincos_tc_kernel(rel_ref, out_ref):
    ph = rel_ref[...] * PHASE_K
    out_ref[:, :HIDDEN] = jnp.cos(ph)
    out_ref[:, HIDDEN:] = jnp.sin(ph)


_sincos_table = pl.pallas_call(
    _sincos_tc_kernel,
    out_shape=jax.ShapeDtypeStruct((NREL, 2 * HIDDEN), jnp.float32),
)


def _sqrt16(v):
    bits = lax.bitcast_convert_type(v, jnp.int32)
    seed = RSQRT_MAGIC - lax.shift_right_arithmetic(bits, 1)
    y = lax.bitcast_convert_type(seed, jnp.float32)
    half_v = 0.5 * v
    for _ in range(2):
        y = y * (1.5 - half_v * y * y)
    return v * y


def _make_sc_kernel():
    mesh = plsc.VectorSubcoreMesh(core_axis_name="c", subcore_axis_name="s")

    @functools.partial(
        pl.kernel,
        mesh=mesh,
        compiler_params=pltpu.CompilerParams(needs_layout_passes=False),
        out_type=jax.ShapeDtypeStruct((BATCH,), jnp.float32),
        scratch_types=[
            pltpu.VMEM((BPW,), jnp.int32),
            pltpu.VMEM((BPW,), jnp.int32),
            pltpu.VMEM((BPW,), jnp.int32),
        ] + [
            pltpu.VMEM((CHUNK, ENT_DIM), jnp.float32)
            for _ in range(6)
        ] + [
            pltpu.VMEM((CHUNK,), jnp.float32),
            pltpu.VMEM((CHUNK,), jnp.float32),
            pltpu.VMEM((16 * RED_STRIDE,), jnp.float32),
            pltpu.SemaphoreType.DMA,
            pltpu.SemaphoreType.DMA,
            pltpu.SemaphoreType.DMA,
        ] + [
            pltpu.SemaphoreType.DMA
            for _ in range(6)
        ] + [
            pltpu.SemaphoreType.DMA,
            pltpu.SemaphoreType.DMA,
        ],
    )
    def rotate_score(sample_hbm, et_hbm, ent_hbm, rel_hbm, out_hbm,
                     hidx_all, tidx_all, eidx_all,
                     head0, tail0, rel0, head1, tail1, rel1,
                     out_v0, out_v1, red_v, sem_ih, sem_it, sem_ir,
                     sem_h0, sem_t0, sem_r0, sem_h1, sem_t1, sem_r1,
                     sem_o0, sem_o1):
        wid = lax.axis_index("s") * 2 + lax.axis_index("c")
        base = wid * BPW
        lane_iota = lax.broadcasted_iota(jnp.int32, (16,), 0)
        col_base = lane_iota * RED_STRIDE

        ci = pltpu.async_copy(
            sample_hbm.at[0, pl.ds(base, BPW)], hidx_all, sem_ih)
        ct = pltpu.async_copy(
            sample_hbm.at[1, pl.ds(base, BPW)], tidx_all, sem_it)
        ce = pltpu.async_copy(et_hbm.at[pl.ds(base, BPW)], eidx_all, sem_ir)
        ci.wait()
        ct.wait()
        ce.wait()

        bufs = (
            (head0, tail0, rel0, sem_h0, sem_t0, sem_r0),
            (head1, tail1, rel1, sem_h1, sem_t1, sem_r1),
        )
        out_bufs = ((out_v0, sem_o0), (out_v1, sem_o1))
        pltpu.async_copy(out_hbm.at[pl.ds(base, CHUNK)], out_v0, sem_o0)
        pltpu.async_copy(out_hbm.at[pl.ds(base, CHUNK)], out_v1, sem_o1)

        def issue(c, b):
            head_v, tail_v, rel_v, sh, st, sr = bufs[b]
            co = c * CHUNK
            pltpu.async_copy(
                ent_hbm.at[hidx_all.at[pl.ds(co, CHUNK)]], head_v, sh)
            pltpu.async_copy(
                ent_hbm.at[tidx_all.at[pl.ds(co, CHUNK)]], tail_v, st)
            pltpu.async_copy(
                rel_hbm.at[eidx_all.at[pl.ds(co, CHUNK)]], rel_v, sr)

        def drain(b):
            head_v, tail_v, rel_v, sh, st, sr = bufs[b]
            pltpu.make_async_copy(
                ent_hbm.at[pl.ds(0, CHUNK)], head_v, sh).wait()
            pltpu.make_async_copy(
                ent_hbm.at[pl.ds(0, CHUNK)], tail_v, st).wait()
            pltpu.make_async_copy(
                rel_hbm.at[pl.ds(0, CHUNK)], rel_v, sr).wait()

        def compute(c, b):
            head_v, tail_v, rel_v, _, _, _ = bufs[b]
            out_v, sem_o = out_bufs[b]
            cb = base + c * CHUNK
            pltpu.make_async_copy(
                out_hbm.at[pl.ds(0, CHUNK)], out_v, sem_o).wait()

            def group_body(g, carry1):
                def elem_body(e, carry2):
                    i = g * 16 + e
                    acc = jnp.zeros((16,), jnp.float32)
                    for dc in range(NDC):
                        sl = pl.ds(dc * 16, 16)
                        sl_im = pl.ds(HIDDEN + dc * 16, 16)
                        cos_r = rel_v[i, sl]
                        sin_r = rel_v[i, sl_im]
                        re_h = head_v[i, sl]
                        im_h = head_v[i, sl_im]
                        re_s = re_h * cos_r - im_h * sin_r - tail_v[i, sl]
                        im_s = re_h * sin_r + im_h * cos_r - tail_v[i, sl_im]
                        acc = acc + _sqrt16(re_s * re_s + im_s * im_s)
                    plsc.store_scatter(red_v, [lane_iota + e * RED_STRIDE],
                                       acc)
                    return carry2

                lax.fori_loop(0, 16, elem_body, 0)
                tot = jnp.zeros((16,), jnp.float32)
                for cc in range(16):
                    tot = tot + plsc.load_gather(red_v, [col_base + cc])
                sc = GAMMA - tot
                u = jnp.exp(-jnp.abs(sc))
                t = u / (u + 2.0)
                t2 = t * t
                log1p = 2.0 * t * (1.0 + t2 * (1.0 / 3.0 + t2 * (
                    1.0 / 5.0 + t2 * (1.0 / 7.0))))
                out_v[pl.ds(g * 16, 16)] = jnp.minimum(sc, 0.0) - log1p
                return carry1

            lax.fori_loop(0, NGRP, group_body, 0)
            pltpu.async_copy(out_v, out_hbm.at[pl.ds(cb, CHUNK)], sem_o)

        issue(0, 0)

        def pair_body(k, carry):
            c0 = 2 * k
            issue(c0 + 1, 1)
            drain(0)
            compute(c0, 0)
            issue(c0 + 2, 0)
            drain(1)
            compute(c0 + 1, 1)
            return carry

        lax.fori_loop(0, NPAIR - 1, pair_body, 0)
        issue(NCHUNK - 1, 1)
        drain(0)
        compute(NCHUNK - 2, 0)
        drain(1)
        compute(NCHUNK - 1, 1)
        pltpu.make_async_copy(
            out_v0, out_hbm.at[pl.ds(base, CHUNK)], sem_o0).wait()
        pltpu.make_async_copy(
            out_v1, out_hbm.at[pl.ds(base, CHUNK)], sem_o1).wait()

    return rotate_score


_SC_KERNEL = _make_sc_kernel()


@jax.jit
def kernel(sample, et, entity_embedding, relation_embedding):
    sincos = _sincos_table(relation_embedding)
    return _SC_KERNEL(sample, et, entity_embedding, sincos)
